# baseline (speedup 1.0000x reference)
import functools

import jax
import jax.numpy as jnp
from jax import lax
from jax.experimental import pallas as pl
from jax.experimental.pallas import tpu as pltpu
from jax.experimental.pallas import tpu_sc as plsc

NC = 2
NS = 16
NW = NC * NS
L = 16


CAP = 88


def _make_degree_kernel(n_nodes, n_blocks):
    mesh = plsc.VectorSubcoreMesh(core_axis_name="c", subcore_axis_name="s")

    @functools.partial(
        pl.kernel, mesh=mesh,
        compiler_params=pltpu.CompilerParams(needs_layout_passes=False),
        out_type=jax.ShapeDtypeStruct((2 * NW, 1, n_nodes), jnp.int32),
        scratch_types=[
            pltpu.VMEM((1, n_nodes), jnp.int32),
            pltpu.VMEM((1, n_nodes), jnp.int32),
            pltpu.VMEM((CAP * 128,), jnp.int32),
            pltpu.VMEM((CAP * 128,), jnp.int32),
        ],
    )
    def deg_kernel(ei_hbm, zero_hbm, out_hbm, cnt_s, cnt_d, sbuf, dbuf):
        n_e = n_blocks * 128
        wid = lax.axis_index("s") * NC + lax.axis_index("c")
        lo = wid * n_blocks // NW
        hi = (wid + 1) * n_blocks // NW
        sblk0 = jnp.minimum(lo, n_blocks - CAP)
        pltpu.sync_copy(ei_hbm.at[pl.ds(sblk0 * 128, CAP * 128)], sbuf)
        pltpu.sync_copy(ei_hbm.at[pl.ds(n_e + sblk0 * 128, CAP * 128)], dbuf)
        pltpu.sync_copy(zero_hbm, cnt_s)
        pltpu.sync_copy(zero_hbm, cnt_d)
        ones = jnp.ones((L,), jnp.int32)
        zidx = jnp.zeros((L,), jnp.int32)
        base = (lo - sblk0) * 128

        def body(i, carry):
            off = base + i * 4 * L
            for u in range(4):
                o = off + u * L
                plsc.addupdate_scatter(cnt_s, [zidx, sbuf[pl.ds(o, L)]],
                                       ones)
                plsc.addupdate_scatter(cnt_d, [zidx, dbuf[pl.ds(o, L)]],
                                       ones)
            return carry

        lax.fori_loop(0, (hi - lo) * (128 // (4 * L)), body, 0)
        pltpu.sync_copy(cnt_s, out_hbm.at[wid])
        pltpu.sync_copy(cnt_d, out_hbm.at[NW + wid])

    return deg_kernel


def _make_scatter_kernel(n_nodes, n_pad, n_blocks):
    rows_per_tile = n_pad // NS
    mesh = plsc.VectorSubcoreMesh(core_axis_name="c", subcore_axis_name="s")

    @functools.partial(
        pl.kernel, mesh=mesh,
        compiler_params=pltpu.CompilerParams(needs_layout_passes=False),
        out_type=jax.ShapeDtypeStruct((2 * n_pad, 128), jnp.float32),
        scratch_types=[
            pltpu.VMEM((4, 128), jnp.int32),
            pltpu.VMEM((3, 128), jnp.int32),
            pltpu.VMEM((3, 128, 128), jnp.float32),
            pltpu.VMEM_SHARED((n_pad, 128), jnp.float32),
            pltpu.SemaphoreType.DMA((4,)),
            pltpu.SemaphoreType.DMA((3,)),
            pltpu.SemaphoreType.DMA((3,)),
        ],
    )
    def msg_kernel(feat_hbm, ei_hbm, zrows_hbm, out_hbm,
                   sidx, didx, rows, agg_sh, ssem, dsem, gsem):
        n_e = n_blocks * 128
        cid = lax.axis_index("c")
        sid = lax.axis_index("s")
        wid = sid * NC + cid
        lo = wid * n_blocks // NW
        hi = (wid + 1) * n_blocks // NW
        nloc = hi - lo

        def scopy(j):
            m = lax.rem(j, 4)
            return pltpu.make_async_copy(
                ei_hbm.at[pl.ds((lo + j) * 128, 128)], sidx.at[m],
                ssem.at[m])

        def dcopy(j):
            m = lax.rem(j, 3)
            return pltpu.make_async_copy(
                ei_hbm.at[pl.ds(n_e + (lo + j) * 128, 128)], didx.at[m],
                dsem.at[m])

        def gath(j):
            k = lax.rem(j, 3)
            return pltpu.make_async_copy(
                feat_hbm.at[sidx.at[lax.rem(j, 4)]], rows.at[k], gsem.at[k])

        row0 = sid * rows_per_tile
        pltpu.sync_copy(zrows_hbm, agg_sh.at[pl.ds(row0, rows_per_tile)])
        plsc.subcore_barrier()

        scopy(0).start()
        scopy(1).start()
        scopy(2).start()
        dcopy(0).start()
        dcopy(1).start()
        scopy(0).wait()
        gath(0).start()
        scopy(1).wait()
        gath(1).start()

        def body(j, carry):
            @pl.when(j + 3 < nloc)
            def _():
                scopy(j + 3).start()

            @pl.when(j + 2 < nloc)
            def _():
                dcopy(j + 2).start()
                scopy(j + 2).wait()
                gath(j + 2).start()

            gath(j).wait()
            dcopy(j).wait()
            pltpu.sync_copy(rows.at[lax.rem(j, 3)],
                            agg_sh.at[didx.at[lax.rem(j, 3)]], add=True)
            return carry

        lax.fori_loop(0, nloc, body, 0)
        plsc.subcore_barrier()
        pltpu.sync_copy(agg_sh.at[pl.ds(row0, rows_per_tile)],
                        out_hbm.at[pl.ds(cid * n_pad + row0, rows_per_tile)])

    return msg_kernel


def _feat_body(cnt_ref, h_ref, feat_ref, s_in_ref):
    out_deg = jnp.sum(cnt_ref[:NW, 0, :], axis=0)
    s = lax.rsqrt(jnp.maximum(out_deg, 1).astype(jnp.float32))
    feat_ref[...] = h_ref[...] * s[:, None]
    in_deg = jnp.sum(cnt_ref[NW:, 0, :], axis=0)
    s_in_ref[...] = lax.rsqrt(
        jnp.maximum(in_deg, 1).astype(jnp.float32))[None, :]


def _scale_feat(cnt3, h):
    n, d = h.shape
    return pl.pallas_call(
        _feat_body,
        out_shape=(jax.ShapeDtypeStruct((n, d), jnp.float32),
                   jax.ShapeDtypeStruct((1, n), jnp.float32)),
    )(cnt3, h)


def _gh_body(h_ref, w_hh_ref, b_hh_ref, gh_ref):
    dn = (((1,), (1,)), ((), ()))
    gh = lax.dot_general(h_ref[...], w_hh_ref[...], dn,
                         preferred_element_type=jnp.float32) + b_hh_ref[...]
    gh_ref[...] = gh.astype(jnp.bfloat16)


def _gru_hside(h, w_hh, b_hh):
    n, d = h.shape
    return pl.pallas_call(
        _gh_body,
        out_shape=jax.ShapeDtypeStruct((n, 3 * d), jnp.bfloat16),
    )(h, w_hh, b_hh)


def _dense_body(n_pad, aggp_ref, s_in_ref, h_ref, w_gcn_ref, b_gcn_ref,
                w_ih_ref, b_ih_ref, gh_ref, out_ref):
    n, d = h_ref.shape
    s_in = s_in_ref[0, :]
    agg = (aggp_ref[pl.ds(0, n), :]
           + aggp_ref[pl.ds(n_pad, n), :]) * s_in[:, None]
    x = jnp.dot(agg, w_gcn_ref[...],
                preferred_element_type=jnp.float32) + b_gcn_ref[...]
    h = h_ref[...]
    dn = (((1,), (1,)), ((), ()))
    gi = lax.dot_general(x, w_ih_ref[...], dn,
                         preferred_element_type=jnp.float32) + b_ih_ref[...]
    gh = gh_ref[...].astype(jnp.float32)
    r = jax.nn.sigmoid(gi[:, :d] + gh[:, :d])
    z = jax.nn.sigmoid(gi[:, d:2 * d] + gh[:, d:2 * d])
    n = jnp.tanh(gi[:, 2 * d:] + r * gh[:, 2 * d:])
    h_new = (1.0 - z) * n + z * h
    out_ref[...] = jnp.where(h_new > 0, h_new, jnp.exp(h_new) - 1.0)


def _dense_gru(n_pad, aggp, s_in, h, w_gcn, b_gcn, w_ih, b_ih, gh):
    n, d = h.shape
    return pl.pallas_call(
        functools.partial(_dense_body, n_pad),
        out_shape=jax.ShapeDtypeStruct((n, d), jnp.float32),
    )(aggp, s_in, h, w_gcn, b_gcn, w_ih, b_ih, gh)


def kernel(h, edge_index, W_gcn, b_gcn, w_ih, w_hh, b_ih, b_hh):
    n, d = h.shape
    e = edge_index.shape[1]
    n_blocks = e // 128
    ei1d = edge_index.astype(jnp.int32).reshape(2 * e)

    zcnt = jnp.zeros((1, n), jnp.int32)
    cnt3 = _make_degree_kernel(n, n_blocks)(ei1d, zcnt)

    feat, s_in = _scale_feat(cnt3, h)

    n_pad = ((n + 8 * NS - 1) // (8 * NS)) * (8 * NS)
    zrows = jnp.zeros((n_pad // NS, d), jnp.float32)
    aggp = _make_scatter_kernel(n, n_pad, n_blocks)(feat, ei1d, zrows)

    gh = _gru_hside(h, w_hh, b_hh.reshape(1, 3 * d))
    return _dense_gru(n_pad, aggp, s_in, h, W_gcn, b_gcn.reshape(1, d),
                      w_ih, b_ih.reshape(1, 3 * d), gh)


# speedup vs baseline: 1.0216x; 1.0216x over previous
import functools

import jax
import jax.numpy as jnp
from jax import lax
from jax.experimental import pallas as pl
from jax.experimental.pallas import tpu as pltpu
from jax.experimental.pallas import tpu_sc as plsc

NC = 2
NS = 16
NW = NC * NS
L = 16


CAP = 88


def _make_degree_kernel(n_nodes, n_blocks):
    mesh = plsc.VectorSubcoreMesh(core_axis_name="c", subcore_axis_name="s")

    @functools.partial(
        pl.kernel, mesh=mesh,
        compiler_params=pltpu.CompilerParams(needs_layout_passes=False),
        out_type=jax.ShapeDtypeStruct((2 * NW, 1, n_nodes), jnp.int32),
        scratch_types=[
            pltpu.VMEM((1, n_nodes), jnp.int32),
            pltpu.VMEM((1, n_nodes), jnp.int32),
            pltpu.VMEM((CAP * 128,), jnp.int32),
            pltpu.VMEM((CAP * 128,), jnp.int32),
        ],
    )
    def deg_kernel(ei_hbm, zero_hbm, out_hbm, cnt_s, cnt_d, sbuf, dbuf):
        n_e = n_blocks * 128
        wid = lax.axis_index("s") * NC + lax.axis_index("c")
        lo = wid * n_blocks // NW
        hi = (wid + 1) * n_blocks // NW
        sblk0 = jnp.minimum(lo, n_blocks - CAP)
        pltpu.sync_copy(ei_hbm.at[pl.ds(sblk0 * 128, CAP * 128)], sbuf)
        pltpu.sync_copy(ei_hbm.at[pl.ds(n_e + sblk0 * 128, CAP * 128)], dbuf)
        pltpu.sync_copy(zero_hbm, cnt_s)
        pltpu.sync_copy(zero_hbm, cnt_d)
        ones = jnp.ones((L,), jnp.int32)
        zidx = jnp.zeros((L,), jnp.int32)
        base = (lo - sblk0) * 128

        def body(i, carry):
            off = base + i * 4 * L
            for u in range(4):
                o = off + u * L
                plsc.addupdate_scatter(cnt_s, [zidx, sbuf[pl.ds(o, L)]],
                                       ones)
                plsc.addupdate_scatter(cnt_d, [zidx, dbuf[pl.ds(o, L)]],
                                       ones)
            return carry

        lax.fori_loop(0, (hi - lo) * (128 // (4 * L)), body, 0)
        pltpu.sync_copy(cnt_s, out_hbm.at[wid])
        pltpu.sync_copy(cnt_d, out_hbm.at[NW + wid])

    return deg_kernel


def _make_scatter_kernel(n_nodes, n_pad, n_blocks):
    rows_per_tile = n_pad // NS
    mesh = plsc.VectorSubcoreMesh(core_axis_name="c", subcore_axis_name="s")

    @functools.partial(
        pl.kernel, mesh=mesh,
        compiler_params=pltpu.CompilerParams(needs_layout_passes=False),
        out_type=jax.ShapeDtypeStruct((2 * n_pad, 128), jnp.float32),
        scratch_types=[
            pltpu.VMEM((4, 128), jnp.int32),
            pltpu.VMEM((3, 128), jnp.int32),
            pltpu.VMEM((3, 128, 128), jnp.float32),
            pltpu.VMEM_SHARED((n_pad, 128), jnp.float32),
            pltpu.SemaphoreType.DMA((4,)),
            pltpu.SemaphoreType.DMA((3,)),
            pltpu.SemaphoreType.DMA((3,)),
        ],
    )
    def msg_kernel(feat_hbm, ei_hbm, zrows_hbm, out_hbm,
                   sidx, didx, rows, agg_sh, ssem, dsem, gsem):
        n_e = n_blocks * 128
        cid = lax.axis_index("c")
        sid = lax.axis_index("s")
        wid = sid * NC + cid
        lo = wid * n_blocks // NW
        hi = (wid + 1) * n_blocks // NW
        nloc = hi - lo

        def scopy(j):
            m = lax.rem(j, 4)
            return pltpu.make_async_copy(
                ei_hbm.at[pl.ds((lo + j) * 128, 128)], sidx.at[m],
                ssem.at[m])

        def dcopy(j):
            m = lax.rem(j, 3)
            return pltpu.make_async_copy(
                ei_hbm.at[pl.ds(n_e + (lo + j) * 128, 128)], didx.at[m],
                dsem.at[m])

        def gath(j):
            k = lax.rem(j, 3)
            return pltpu.make_async_copy(
                feat_hbm.at[sidx.at[lax.rem(j, 4)]], rows.at[k], gsem.at[k])

        row0 = sid * rows_per_tile
        pltpu.sync_copy(zrows_hbm, agg_sh.at[pl.ds(row0, rows_per_tile)])
        plsc.subcore_barrier()

        scopy(0).start()
        scopy(1).start()
        scopy(2).start()
        dcopy(0).start()
        dcopy(1).start()
        scopy(0).wait()
        gath(0).start()
        scopy(1).wait()
        gath(1).start()

        def body(j, carry):
            @pl.when(j + 3 < nloc)
            def _():
                scopy(j + 3).start()

            @pl.when(j + 2 < nloc)
            def _():
                dcopy(j + 2).start()
                scopy(j + 2).wait()
                gath(j + 2).start()

            gath(j).wait()
            dcopy(j).wait()
            pltpu.sync_copy(rows.at[lax.rem(j, 3)],
                            agg_sh.at[didx.at[lax.rem(j, 3)]], add=True)
            return carry

        lax.fori_loop(0, nloc, body, 0)
        plsc.subcore_barrier()
        pltpu.sync_copy(agg_sh.at[pl.ds(row0, rows_per_tile)],
                        out_hbm.at[pl.ds(cid * n_pad + row0, rows_per_tile)])

    return msg_kernel


def _feat_body(cnt_ref, h_ref, feat_ref, s_in_ref):
    out_deg = jnp.sum(cnt_ref[:NW, 0, :], axis=0)
    s = lax.rsqrt(jnp.maximum(out_deg, 1).astype(jnp.float32))
    feat_ref[...] = h_ref[...] * s[:, None]
    in_deg = jnp.sum(cnt_ref[NW:, 0, :], axis=0)
    s_in_ref[...] = lax.rsqrt(
        jnp.maximum(in_deg, 1).astype(jnp.float32))[None, :]


def _scale_feat(cnt3, h):
    n, d = h.shape
    return pl.pallas_call(
        _feat_body,
        out_shape=(jax.ShapeDtypeStruct((n, d), jnp.float32),
                   jax.ShapeDtypeStruct((1, n), jnp.float32)),
    )(cnt3, h)


def _dense_body(n_pad, aggp_ref, s_in_ref, h_ref, w_gcn_ref, b_gcn_ref,
                w_ih_ref, w_hh_ref, b_ih_ref, b_hh_ref, out_ref):
    n, d = h_ref.shape
    s_in = s_in_ref[0, :]
    agg = (aggp_ref[pl.ds(0, n), :]
           + aggp_ref[pl.ds(n_pad, n), :]) * s_in[:, None]
    x = jnp.dot(agg, w_gcn_ref[...],
                preferred_element_type=jnp.float32) + b_gcn_ref[...]
    h = h_ref[...]
    dn = (((1,), (1,)), ((), ()))
    gi = lax.dot_general(x, w_ih_ref[...], dn,
                         preferred_element_type=jnp.float32) + b_ih_ref[...]
    gh = lax.dot_general(h, w_hh_ref[...], dn,
                         preferred_element_type=jnp.float32) + b_hh_ref[...]
    r = jax.nn.sigmoid(gi[:, :d] + gh[:, :d])
    z = jax.nn.sigmoid(gi[:, d:2 * d] + gh[:, d:2 * d])
    n = jnp.tanh(gi[:, 2 * d:] + r * gh[:, 2 * d:])
    h_new = (1.0 - z) * n + z * h
    out_ref[...] = jnp.where(h_new > 0, h_new, jnp.exp(h_new) - 1.0)


def _dense_gru(n_pad, aggp, s_in, h, w_gcn, b_gcn, w_ih, w_hh, b_ih, b_hh):
    n, d = h.shape
    return pl.pallas_call(
        functools.partial(_dense_body, n_pad),
        out_shape=jax.ShapeDtypeStruct((n, d), jnp.float32),
    )(aggp, s_in, h, w_gcn, b_gcn, w_ih, w_hh, b_ih, b_hh)


def kernel(h, edge_index, W_gcn, b_gcn, w_ih, w_hh, b_ih, b_hh):
    n, d = h.shape
    e = edge_index.shape[1]
    n_blocks = e // 128
    ei1d = edge_index.astype(jnp.int32).reshape(2 * e)

    zcnt = jnp.zeros((1, n), jnp.int32)
    cnt3 = _make_degree_kernel(n, n_blocks)(ei1d, zcnt)

    feat, s_in = _scale_feat(cnt3, h)

    n_pad = ((n + 8 * NS - 1) // (8 * NS)) * (8 * NS)
    zrows = jnp.zeros((n_pad // NS, d), jnp.float32)
    aggp = _make_scatter_kernel(n, n_pad, n_blocks)(feat, ei1d, zrows)

    return _dense_gru(n_pad, aggp, s_in, h, W_gcn, b_gcn.reshape(1, d),
                      w_ih, w_hh, b_ih.reshape(1, 3 * d),
                      b_hh.reshape(1, 3 * d))


# speedup vs baseline: 1.0639x; 1.0414x over previous
import functools

import jax
import jax.numpy as jnp
from jax import lax
from jax.experimental import pallas as pl
from jax.experimental.pallas import tpu as pltpu
from jax.experimental.pallas import tpu_sc as plsc

NC = 2
NS = 16
NW = NC * NS
L = 16


CAP = 88


def _make_degree_kernel(n_nodes, n_blocks):
    mesh = plsc.VectorSubcoreMesh(core_axis_name="c", subcore_axis_name="s")

    @functools.partial(
        pl.kernel, mesh=mesh,
        compiler_params=pltpu.CompilerParams(needs_layout_passes=False),
        out_type=jax.ShapeDtypeStruct((2 * NW, 1, n_nodes), jnp.int32),
        scratch_types=[
            pltpu.VMEM((1, n_nodes), jnp.int32),
            pltpu.VMEM((1, n_nodes), jnp.int32),
            pltpu.VMEM((2, CAP * 128), jnp.int32),
        ],
    )
    def deg_kernel(ei_hbm, zero_hbm, out_hbm, cnt_s, cnt_d, sdbuf):
        wid = lax.axis_index("s") * NC + lax.axis_index("c")
        lo = wid * n_blocks // NW
        hi = (wid + 1) * n_blocks // NW
        sblk0 = jnp.minimum(lo, n_blocks - CAP)
        pltpu.sync_copy(ei_hbm.at[:, pl.ds(sblk0 * 128, CAP * 128)], sdbuf)
        pltpu.sync_copy(zero_hbm, cnt_s)
        pltpu.sync_copy(zero_hbm, cnt_d)
        ones = jnp.ones((L,), jnp.int32)
        zidx = jnp.zeros((L,), jnp.int32)
        base = (lo - sblk0) * 128

        def body(i, carry):
            off = base + i * 4 * L
            for u in range(4):
                o = off + u * L
                plsc.addupdate_scatter(cnt_s, [zidx, sdbuf[0, pl.ds(o, L)]],
                                       ones)
                plsc.addupdate_scatter(cnt_d, [zidx, sdbuf[1, pl.ds(o, L)]],
                                       ones)
            return carry

        lax.fori_loop(0, (hi - lo) * (128 // (4 * L)), body, 0)
        pltpu.sync_copy(cnt_s, out_hbm.at[wid])
        pltpu.sync_copy(cnt_d, out_hbm.at[NW + wid])

    return deg_kernel


def _make_scatter_kernel(n_nodes, n_blocks):
    rpt = (n_nodes // NS + 7) // 8 * 8
    rlast = n_nodes - (NS - 1) * rpt
    mesh = plsc.VectorSubcoreMesh(core_axis_name="c", subcore_axis_name="s")

    @functools.partial(
        pl.kernel, mesh=mesh,
        compiler_params=pltpu.CompilerParams(needs_layout_passes=False),
        out_type=jax.ShapeDtypeStruct((2 * n_nodes, 128), jnp.float32),
        scratch_types=[
            pltpu.VMEM((4, 2, 128), jnp.int32),
            pltpu.VMEM((3, 128, 128), jnp.float32),
            pltpu.VMEM_SHARED((n_nodes, 128), jnp.float32),
            pltpu.SemaphoreType.DMA((4,)),
            pltpu.SemaphoreType.DMA((3,)),
        ],
    )
    def msg_kernel(feat_hbm, ei_hbm, zrows_hbm, out_hbm,
                   sdidx, rows, agg_sh, isem, gsem):
        cid = lax.axis_index("c")
        sid = lax.axis_index("s")
        wid = sid * NC + cid
        lo = wid * n_blocks // NW
        hi = (wid + 1) * n_blocks // NW
        nloc = hi - lo

        def icopy(j):
            m = lax.rem(j, 4)
            return pltpu.make_async_copy(
                ei_hbm.at[:, pl.ds((lo + j) * 128, 128)], sdidx.at[m],
                isem.at[m])

        def gath(j):
            k = lax.rem(j, 3)
            return pltpu.make_async_copy(
                feat_hbm.at[sdidx.at[lax.rem(j, 4), 0]], rows.at[k],
                gsem.at[k])

        row0 = sid * rpt

        @pl.when(sid < NS - 1)
        def _():
            pltpu.sync_copy(zrows_hbm, agg_sh.at[pl.ds(row0, rpt)])

        @pl.when(sid == NS - 1)
        def _():
            pltpu.sync_copy(zrows_hbm.at[pl.ds(0, rlast)],
                            agg_sh.at[pl.ds(row0, rlast)])

        plsc.subcore_barrier()

        icopy(0).start()
        icopy(1).start()
        icopy(2).start()
        icopy(0).wait()
        gath(0).start()
        icopy(1).wait()
        gath(1).start()

        def body(j, carry):
            @pl.when(j + 3 < nloc)
            def _():
                icopy(j + 3).start()

            @pl.when(j + 2 < nloc)
            def _():
                icopy(j + 2).wait()
                gath(j + 2).start()

            gath(j).wait()
            pltpu.sync_copy(rows.at[lax.rem(j, 3)],
                            agg_sh.at[sdidx.at[lax.rem(j, 4), 1]], add=True)
            return carry

        lax.fori_loop(0, nloc, body, 0)
        plsc.subcore_barrier()

        @pl.when(sid < NS - 1)
        def _():
            pltpu.sync_copy(agg_sh.at[pl.ds(row0, rpt)],
                            out_hbm.at[pl.ds(cid * n_nodes + row0, rpt)])

        @pl.when(sid == NS - 1)
        def _():
            pltpu.sync_copy(agg_sh.at[pl.ds(row0, rlast)],
                            out_hbm.at[pl.ds(cid * n_nodes + row0, rlast)])

    return msg_kernel


def _feat_body(cnt_ref, h_ref, feat_ref, s_in_ref):
    out_deg = jnp.sum(cnt_ref[:NW, 0, :], axis=0)
    s = lax.rsqrt(jnp.maximum(out_deg, 1).astype(jnp.float32))
    feat_ref[...] = h_ref[...] * s[:, None]
    in_deg = jnp.sum(cnt_ref[NW:, 0, :], axis=0)
    s_in_ref[...] = lax.rsqrt(
        jnp.maximum(in_deg, 1).astype(jnp.float32))[None, :]


def _scale_feat(cnt3, h):
    n, d = h.shape
    return pl.pallas_call(
        _feat_body,
        out_shape=(jax.ShapeDtypeStruct((n, d), jnp.float32),
                   jax.ShapeDtypeStruct((1, n), jnp.float32)),
    )(cnt3, h)


def _dense_body(n_pad, aggp_ref, s_in_ref, h_ref, w_gcn_ref, b_gcn_ref,
                w_ih_ref, w_hh_ref, b_ih_ref, b_hh_ref, out_ref):
    n, d = h_ref.shape
    s_in = s_in_ref[0, :]
    agg = (aggp_ref[pl.ds(0, n), :]
           + aggp_ref[pl.ds(n_pad, n), :]) * s_in[:, None]
    x = jnp.dot(agg, w_gcn_ref[...],
                preferred_element_type=jnp.float32) + b_gcn_ref[...]
    h = h_ref[...]
    dn = (((1,), (1,)), ((), ()))
    gi = lax.dot_general(x, w_ih_ref[...], dn,
                         preferred_element_type=jnp.float32) + b_ih_ref[...]
    gh = lax.dot_general(h, w_hh_ref[...], dn,
                         preferred_element_type=jnp.float32) + b_hh_ref[...]
    r = jax.nn.sigmoid(gi[:, :d] + gh[:, :d])
    z = jax.nn.sigmoid(gi[:, d:2 * d] + gh[:, d:2 * d])
    n = jnp.tanh(gi[:, 2 * d:] + r * gh[:, 2 * d:])
    h_new = (1.0 - z) * n + z * h
    out_ref[...] = jnp.where(h_new > 0, h_new, jnp.exp(h_new) - 1.0)


def _dense_gru(n_pad, aggp, s_in, h, w_gcn, b_gcn, w_ih, w_hh, b_ih, b_hh):
    n, d = h.shape
    return pl.pallas_call(
        functools.partial(_dense_body, n_pad),
        out_shape=jax.ShapeDtypeStruct((n, d), jnp.float32),
    )(aggp, s_in, h, w_gcn, b_gcn, w_ih, w_hh, b_ih, b_hh)


def kernel(h, edge_index, W_gcn, b_gcn, w_ih, w_hh, b_ih, b_hh):
    n, d = h.shape
    e = edge_index.shape[1]
    n_blocks = e // 128
    ei32 = edge_index.astype(jnp.int32)

    zcnt = jnp.zeros((1, n), jnp.int32)
    cnt3 = _make_degree_kernel(n, n_blocks)(ei32, zcnt)

    feat, s_in = _scale_feat(cnt3, h)

    rpt = (n // NS + 7) // 8 * 8
    zrows = jnp.zeros((rpt, d), jnp.float32)
    aggp = _make_scatter_kernel(n, n_blocks)(feat, ei32, zrows)

    return _dense_gru(n, aggp, s_in, h, W_gcn, b_gcn.reshape(1, d),
                      w_ih, w_hh, b_ih.reshape(1, 3 * d),
                      b_hh.reshape(1, 3 * d))
